# Initial kernel scaffold; baseline (speedup 1.0000x reference)
import jax
import jax.numpy as jnp
from jax.experimental import pallas as pl


def kernel(x, neutral_idx, table_w):
    raise NotImplementedError("write your pallas kernel here")


# speedup vs baseline: 91.9269x; 91.9269x over previous
import functools

import jax
import jax.numpy as jnp
from jax import lax
from jax.experimental import pallas as pl
from jax.experimental.pallas import tpu as pltpu
from jax.experimental.pallas import tpu_sc as plsc

LN2 = 0.6931471805599453
CHUNK = 2048
GATHER = 128
LANES = 16


@functools.lru_cache(maxsize=None)
def _build(n_total: int, nc: int, ns: int):
    nw = nc * ns
    per_w = n_total // nw
    assert per_w * nw == n_total and per_w % CHUNK == 0
    n_chunks = per_w // CHUNK
    mesh = plsc.VectorSubcoreMesh(
        core_axis_name="c", subcore_axis_name="s", num_cores=nc, num_subcores=ns
    )

    def body(table_hbm, xf_hbm, nidx_hbm, out_hbm, idx_v, vals_v, nidx_v, nval_v, sem):
        wid = lax.axis_index("s") * nc + lax.axis_index("c")
        base0 = wid * per_w

        pltpu.sync_copy(nidx_hbm, nidx_v)
        pltpu.async_copy(table_hbm.at[nidx_v], nval_v, sem).wait()

        def chunk_body(g, carry):
            base = base0 + g * CHUNK
            pltpu.sync_copy(xf_hbm.at[pl.ds(base, CHUNK)], idx_v)
            copies = [
                pltpu.async_copy(
                    table_hbm.at[idx_v.at[pl.ds(j * GATHER, GATHER)]],
                    vals_v.at[pl.ds(j * GATHER, GATHER)],
                    sem,
                )
                for j in range(CHUNK // GATHER)
            ]
            for cp in copies:
                cp.wait()
            nval = nval_v[...]

            def vec_body(i, c):
                v = vals_v[pl.ds(i * LANES, LANES)]
                vals_v[pl.ds(i * LANES, LANES)] = jnp.exp((v - nval) * LN2)
                return c

            lax.fori_loop(0, CHUNK // LANES, vec_body, 0, unroll=4)
            pltpu.sync_copy(vals_v, out_hbm.at[pl.ds(base, CHUNK)])
            return carry

        lax.fori_loop(0, n_chunks, chunk_body, 0)

    return pl.kernel(
        body,
        out_type=jax.ShapeDtypeStruct((n_total,), jnp.float32),
        mesh=mesh,
        scratch_types=[
            pltpu.VMEM((CHUNK,), jnp.int32),
            pltpu.VMEM((CHUNK,), jnp.float32),
            pltpu.VMEM((LANES,), jnp.int32),
            pltpu.VMEM((LANES,), jnp.float32),
            pltpu.SemaphoreType.DMA,
        ],
    )


def kernel(x, neutral_idx, table_w):
    b, h = x.shape
    n_total = b * h
    info = plsc.get_sparse_core_info()
    table = table_w.reshape(-1)
    xf = x.reshape(-1)
    nidx = jnp.full((LANES,), neutral_idx, dtype=jnp.int32)
    out = _build(n_total, info.num_cores, info.num_subcores)(table, xf, nidx)
    return out.reshape(b, h, 1)


# speedup vs baseline: 133.1024x; 1.4479x over previous
import functools

import jax
import jax.numpy as jnp
from jax import lax
from jax.experimental import pallas as pl
from jax.experimental.pallas import tpu as pltpu
from jax.experimental.pallas import tpu_sc as plsc

LN2 = 0.6931471805599453
CHUNK = 2048
GATHER = 128
LANES = 16


@functools.lru_cache(maxsize=None)
def _build(n_total: int, n_rows: int, nc: int, ns: int):
    nw = nc * ns
    per_w = n_total // nw
    assert per_w * nw == n_total and per_w % CHUNK == 0
    n_chunks = per_w // CHUNK
    mesh = plsc.VectorSubcoreMesh(
        core_axis_name="c", subcore_axis_name="s", num_cores=nc, num_subcores=ns
    )

    def body(table_hbm, xf_hbm, nidx_hbm, out_hbm, idx_v, vals_v, nidx_v, nval_v,
             table_sh, stage_v, sem):
        wid = lax.axis_index("s") * nc + lax.axis_index("c")
        sid = lax.axis_index("s")
        base0 = wid * per_w

        stage = n_rows // 8
        piece = stage // 5

        @pl.when(sid < 8)
        def _():
            def stage_body(p, c):
                off = sid * stage + p * piece
                pltpu.sync_copy(table_hbm.at[pl.ds(off, piece)], stage_v)
                pltpu.sync_copy(stage_v, table_sh.at[pl.ds(off, piece)])
                return c

            lax.fori_loop(0, 5, stage_body, 0)

        pltpu.sync_copy(nidx_hbm, nidx_v)
        pltpu.async_copy(table_hbm.at[nidx_v], nval_v, sem).wait()
        plsc.subcore_barrier()

        def chunk_body(g, carry):
            base = base0 + g * CHUNK
            pltpu.sync_copy(xf_hbm.at[pl.ds(base, CHUNK)], idx_v)
            copies = [
                pltpu.async_copy(
                    table_sh.at[idx_v.at[pl.ds(j * GATHER, GATHER)]],
                    vals_v.at[pl.ds(j * GATHER, GATHER)],
                    sem,
                )
                for j in range(CHUNK // GATHER)
            ]
            for cp in copies:
                cp.wait()
            nval = nval_v[...]

            def vec_body(i, c):
                v = vals_v[pl.ds(i * LANES, LANES)]
                vals_v[pl.ds(i * LANES, LANES)] = jnp.exp((v - nval) * LN2)
                return c

            lax.fori_loop(0, CHUNK // LANES, vec_body, 0, unroll=4)
            pltpu.sync_copy(vals_v, out_hbm.at[pl.ds(base, CHUNK)])
            return carry

        lax.fori_loop(0, n_chunks, chunk_body, 0)

    return pl.kernel(
        body,
        out_type=jax.ShapeDtypeStruct((n_total,), jnp.float32),
        mesh=mesh,
        scratch_types=[
            pltpu.VMEM((CHUNK,), jnp.int32),
            pltpu.VMEM((CHUNK,), jnp.float32),
            pltpu.VMEM((LANES,), jnp.int32),
            pltpu.VMEM((LANES,), jnp.float32),
            pltpu.VMEM_SHARED((n_rows,), jnp.float32),
            pltpu.VMEM((n_rows // 40,), jnp.float32),
            pltpu.SemaphoreType.DMA,
        ],
    )


def kernel(x, neutral_idx, table_w):
    b, h = x.shape
    n_total = b * h
    info = plsc.get_sparse_core_info()
    table = table_w.reshape(-1)
    xf = x.reshape(-1)
    nidx = jnp.full((LANES,), neutral_idx, dtype=jnp.int32)
    out = _build(n_total, table.shape[0], info.num_cores, info.num_subcores)(
        table, xf, nidx
    )
    return out.reshape(b, h, 1)


# speedup vs baseline: 176.8466x; 1.3287x over previous
import functools

import jax
import jax.numpy as jnp
from jax import lax
from jax.experimental import pallas as pl
from jax.experimental.pallas import tpu as pltpu
from jax.experimental.pallas import tpu_sc as plsc

LN2 = 0.6931471805599453
CHUNK = 6400
GATHER = 128
LANES = 16
STAGE_TILES = 8
STAGE_PIECES = 5


@functools.lru_cache(maxsize=None)
def _build(n_total: int, n_rows: int, nc: int, ns: int):
    nw = nc * ns
    per_w = n_total // nw
    assert per_w * nw == n_total and per_w % CHUNK == 0
    n_chunks = per_w // CHUNK
    assert n_chunks % 2 == 0 and n_chunks >= 4
    stage = n_rows // STAGE_TILES
    piece = stage // STAGE_PIECES
    assert stage * STAGE_TILES == n_rows and piece * STAGE_PIECES == stage
    assert piece % 8 == 0 and CHUNK % GATHER == 0 and CHUNK % LANES == 0
    mesh = plsc.VectorSubcoreMesh(
        core_axis_name="c", subcore_axis_name="s", num_cores=nc, num_subcores=ns
    )

    def body(table_hbm, xf_hbm, nidx_hbm, out_hbm,
             idx0, idx1, vals0, vals1, nidx_v, nval_v, table_sh, stage_v,
             gsem0, gsem1, isem0, isem1, osem0, osem1):
        idx = (idx0, idx1)
        vals = (vals0, vals1)
        gsem = (gsem0, gsem1)
        isem = (isem0, isem1)
        osem = (osem0, osem1)
        wid = lax.axis_index("s") * nc + lax.axis_index("c")
        sid = lax.axis_index("s")
        base0 = wid * per_w

        @pl.when(sid < STAGE_TILES)
        def _():
            def stage_body(p, c):
                off = sid * stage + p * piece
                pltpu.sync_copy(table_hbm.at[pl.ds(off, piece)], stage_v)
                pltpu.sync_copy(stage_v, table_sh.at[pl.ds(off, piece)])
                return c

            lax.fori_loop(0, STAGE_PIECES, stage_body, 0)

        pltpu.sync_copy(nidx_hbm, nidx_v)
        pltpu.async_copy(table_hbm.at[nidx_v], nval_v, gsem0).wait()
        plsc.subcore_barrier()

        def fire_gathers(idx_b, vals_b, sem):
            def fg(j, c):
                pltpu.async_copy(
                    table_sh.at[idx_b.at[pl.ds(j * GATHER, GATHER)]],
                    vals_b.at[pl.ds(j * GATHER, GATHER)],
                    sem,
                )
                return c

            lax.fori_loop(0, CHUNK // GATHER, fg, 0)

        def drain(src, dst, sem):
            pltpu.make_async_copy(src, dst, sem).wait()

        def compute(vals_b):
            nval = nval_v[...]

            def vec_body(i, c):
                v = vals_b[pl.ds(i * LANES, LANES)]
                vals_b[pl.ds(i * LANES, LANES)] = jnp.exp((v - nval) * LN2)
                return c

            lax.fori_loop(0, CHUNK // LANES, vec_body, 0, unroll=8)

        hbm_f32 = out_hbm.at[pl.ds(0, CHUNK)]

        def step(g, b):
            ob = 1 - b

            @pl.when(g + 1 < n_chunks)
            def _():
                drain(xf_hbm.at[pl.ds(0, CHUNK)], idx[ob], isem[ob])

                @pl.when(g >= 1)
                def _():
                    drain(vals[ob], hbm_f32, osem[ob])

                fire_gathers(idx[ob], vals[ob], gsem[ob])

            drain(hbm_f32, vals[b], gsem[b])

            @pl.when(g + 2 < n_chunks)
            def _():
                pltpu.async_copy(
                    xf_hbm.at[pl.ds(base0 + (g + 2) * CHUNK, CHUNK)],
                    idx[b], isem[b],
                )

            compute(vals[b])
            pltpu.async_copy(
                vals[b], out_hbm.at[pl.ds(base0 + g * CHUNK, CHUNK)], osem[b]
            )

        pltpu.async_copy(xf_hbm.at[pl.ds(base0, CHUNK)], idx0, isem0)
        pltpu.async_copy(xf_hbm.at[pl.ds(base0 + CHUNK, CHUNK)], idx1, isem1)
        drain(xf_hbm.at[pl.ds(0, CHUNK)], idx0, isem0)
        fire_gathers(idx0, vals0, gsem0)

        def pair(k, c):
            step(2 * k, 0)
            step(2 * k + 1, 1)
            return c

        lax.fori_loop(0, n_chunks // 2, pair, 0)

        drain(vals0, hbm_f32, osem0)
        drain(vals1, hbm_f32, osem1)

    return pl.kernel(
        body,
        out_type=jax.ShapeDtypeStruct((n_total,), jnp.float32),
        mesh=mesh,
        scratch_types=[
            pltpu.VMEM((CHUNK,), jnp.int32),
            pltpu.VMEM((CHUNK,), jnp.int32),
            pltpu.VMEM((CHUNK,), jnp.float32),
            pltpu.VMEM((CHUNK,), jnp.float32),
            pltpu.VMEM((LANES,), jnp.int32),
            pltpu.VMEM((LANES,), jnp.float32),
            pltpu.VMEM_SHARED((n_rows,), jnp.float32),
            pltpu.VMEM((n_rows // (STAGE_TILES * STAGE_PIECES),), jnp.float32),
            pltpu.SemaphoreType.DMA,
            pltpu.SemaphoreType.DMA,
            pltpu.SemaphoreType.DMA,
            pltpu.SemaphoreType.DMA,
            pltpu.SemaphoreType.DMA,
            pltpu.SemaphoreType.DMA,
        ],
    )


def kernel(x, neutral_idx, table_w):
    b, h = x.shape
    n_total = b * h
    info = plsc.get_sparse_core_info()
    table = table_w.reshape(-1)
    xf = x.reshape(-1)
    nidx = jnp.full((LANES,), neutral_idx, dtype=jnp.int32)
    out = _build(n_total, table.shape[0], info.num_cores, info.num_subcores)(
        table, xf, nidx
    )
    return out.reshape(b, h, 1)


# speedup vs baseline: 181.4642x; 1.0261x over previous
import functools

import jax
import jax.numpy as jnp
from jax import lax
from jax.experimental import pallas as pl
from jax.experimental.pallas import tpu as pltpu
from jax.experimental.pallas import tpu_sc as plsc

LN2 = 0.6931471805599453
CHUNK = 5120
GATHER = 512
LANES = 16
STAGE_TILES = 8
STAGE_PIECES = 5


@functools.lru_cache(maxsize=None)
def _build(n_total: int, n_rows: int, nc: int, ns: int):
    nw = nc * ns
    per_w = n_total // nw
    assert per_w * nw == n_total and per_w % CHUNK == 0
    n_chunks = per_w // CHUNK
    assert n_chunks % 2 == 0 and n_chunks >= 4
    stage = n_rows // STAGE_TILES
    piece = stage // STAGE_PIECES
    assert stage * STAGE_TILES == n_rows and piece * STAGE_PIECES == stage
    assert piece % 8 == 0 and CHUNK % GATHER == 0 and CHUNK % LANES == 0
    mesh = plsc.VectorSubcoreMesh(
        core_axis_name="c", subcore_axis_name="s", num_cores=nc, num_subcores=ns
    )

    def body(table_hbm, xf_hbm, nidx_hbm, out_hbm,
             idx0, idx1, vals0, vals1, nidx_v, nval_v, table_sh, stage_v,
             gsem0, gsem1, isem0, isem1, osem0, osem1):
        idx = (idx0, idx1)
        vals = (vals0, vals1)
        gsem = (gsem0, gsem1)
        isem = (isem0, isem1)
        osem = (osem0, osem1)
        wid = lax.axis_index("s") * nc + lax.axis_index("c")
        sid = lax.axis_index("s")
        base0 = wid * per_w

        @pl.when(sid < STAGE_TILES)
        def _():
            def stage_body(p, c):
                off = sid * stage + p * piece
                pltpu.sync_copy(table_hbm.at[pl.ds(off, piece)], stage_v)
                pltpu.sync_copy(stage_v, table_sh.at[pl.ds(off, piece)])
                return c

            lax.fori_loop(0, STAGE_PIECES, stage_body, 0)

        pltpu.sync_copy(nidx_hbm, nidx_v)
        pltpu.async_copy(table_hbm.at[nidx_v], nval_v, gsem0).wait()
        plsc.subcore_barrier()

        def fire_gathers(idx_b, vals_b, sem):
            def fg(j, c):
                pltpu.async_copy(
                    table_sh.at[idx_b.at[pl.ds(j * GATHER, GATHER)]],
                    vals_b.at[pl.ds(j * GATHER, GATHER)],
                    sem,
                )
                return c

            lax.fori_loop(0, CHUNK // GATHER, fg, 0)

        def drain(src, dst, sem):
            pltpu.make_async_copy(src, dst, sem).wait()

        def compute(vals_b):
            nval = nval_v[...]

            def vec_body(i, c):
                v = vals_b[pl.ds(i * LANES, LANES)]
                vals_b[pl.ds(i * LANES, LANES)] = jnp.exp((v - nval) * LN2)
                return c

            lax.fori_loop(0, CHUNK // LANES, vec_body, 0, unroll=8)

        hbm_f32 = out_hbm.at[pl.ds(0, CHUNK)]

        def step(g, b):
            ob = 1 - b

            @pl.when(g + 1 < n_chunks)
            def _():
                drain(xf_hbm.at[pl.ds(0, CHUNK)], idx[ob], isem[ob])

                @pl.when(g >= 1)
                def _():
                    drain(vals[ob], hbm_f32, osem[ob])

                fire_gathers(idx[ob], vals[ob], gsem[ob])

            drain(hbm_f32, vals[b], gsem[b])

            @pl.when(g + 2 < n_chunks)
            def _():
                pltpu.async_copy(
                    xf_hbm.at[pl.ds(base0 + (g + 2) * CHUNK, CHUNK)],
                    idx[b], isem[b],
                )

            compute(vals[b])
            pltpu.async_copy(
                vals[b], out_hbm.at[pl.ds(base0 + g * CHUNK, CHUNK)], osem[b]
            )

        pltpu.async_copy(xf_hbm.at[pl.ds(base0, CHUNK)], idx0, isem0)
        pltpu.async_copy(xf_hbm.at[pl.ds(base0 + CHUNK, CHUNK)], idx1, isem1)
        drain(xf_hbm.at[pl.ds(0, CHUNK)], idx0, isem0)
        fire_gathers(idx0, vals0, gsem0)

        def pair(k, c):
            step(2 * k, 0)
            step(2 * k + 1, 1)
            return c

        lax.fori_loop(0, n_chunks // 2, pair, 0)

        drain(vals0, hbm_f32, osem0)
        drain(vals1, hbm_f32, osem1)

    return pl.kernel(
        body,
        out_type=jax.ShapeDtypeStruct((n_total,), jnp.float32),
        mesh=mesh,
        scratch_types=[
            pltpu.VMEM((CHUNK,), jnp.int32),
            pltpu.VMEM((CHUNK,), jnp.int32),
            pltpu.VMEM((CHUNK,), jnp.float32),
            pltpu.VMEM((CHUNK,), jnp.float32),
            pltpu.VMEM((LANES,), jnp.int32),
            pltpu.VMEM((LANES,), jnp.float32),
            pltpu.VMEM_SHARED((n_rows,), jnp.float32),
            pltpu.VMEM((n_rows // (STAGE_TILES * STAGE_PIECES),), jnp.float32),
            pltpu.SemaphoreType.DMA,
            pltpu.SemaphoreType.DMA,
            pltpu.SemaphoreType.DMA,
            pltpu.SemaphoreType.DMA,
            pltpu.SemaphoreType.DMA,
            pltpu.SemaphoreType.DMA,
        ],
    )


def kernel(x, neutral_idx, table_w):
    b, h = x.shape
    n_total = b * h
    info = plsc.get_sparse_core_info()
    table = table_w.reshape(-1)
    xf = x.reshape(-1)
    nidx = jnp.full((LANES,), neutral_idx, dtype=jnp.int32)
    out = _build(n_total, table.shape[0], info.num_cores, info.num_subcores)(
        table, xf, nidx
    )
    return out.reshape(b, h, 1)
